# baseline (speedup 1.0000x reference)
import jax
import jax.numpy as jnp
from jax import lax
from jax.experimental import pallas as pl
from jax.experimental.pallas import tpu as pltpu
from jax.experimental.pallas import tpu_sc as plsc

B, L, E, O = 16384, 200, 3, 5
NC, NS = 2, 16
NW = NC * NS
ROWS = B // NW
BB = 128
NBB = ROWS // BB
LC = 50
NLC = L // LC
LU = 2


def _build_lut(emb_v, w_v, b_v, lut_v):
    for off in (0, 16):
        p = jnp.minimum(lax.iota(jnp.int32, 16) + off, O * O - 1)
        i_v = p // O
        o_v = p % O
        acc = plsc.load_gather(b_v, [o_v])
        for d in range(E):
            dd = jnp.full((16,), d, jnp.int32)
            ev = plsc.load_gather(emb_v, [i_v, dd])
            wv = plsc.load_gather(w_v, [o_v, dd])
            acc = acc + ev * wv
        for r in range(16):
            rr = jnp.full((16,), r, jnp.int32)
            plsc.store_scatter(lut_v, [rr, p], acc)


def _sc_body(emb_hbm, w_hbm, b_hbm, x_hbm, out_hbm,
             emb_v, w_v, b_v, lut_v, x_st, out_st,
             sem_x0, sem_x1, sem_o0, sem_o1):
    c = lax.axis_index("c")
    s = lax.axis_index("s")
    wid = c * NS + s
    b0 = wid * ROWS

    pltpu.sync_copy(emb_hbm, emb_v)
    pltpu.sync_copy(w_hbm, w_v)
    pltpu.sync_copy(b_hbm, b_v)
    _build_lut(emb_v, w_v, b_v, lut_v)

    sem_x = (sem_x0, sem_x1)
    sem_o = (sem_o0, sem_o1)
    lane = lax.iota(jnp.int32, 16)

    def compute(xref, oref, l0):
        def l_body(it, carry):
            for u in range(LU):
                dl = it * LU + u
                l_v = jnp.full((16,), l0, jnp.int32) + dl
                for g in range(BB // 16):
                    b_v = lane + g * 16
                    xg = plsc.load_gather(xref, [b_v, l_v])
                    base = xg * O
                    for o in range(O):
                        v = plsc.load_gather(lut_v, [lane, base + o])
                        oref[o, dl, pl.ds(g * 16, 16)] = v
            return carry
        lax.fori_loop(0, LC // LU, l_body, 0)

    def start_x(i):
        return pltpu.async_copy(
            x_hbm.at[pl.ds(b0 + i * BB, BB), :],
            x_st.at[i % 2, :, pl.ds(0, L)], sem_x[i % 2])

    hx = [None, None]
    ho = [None, None]
    hx[0] = start_x(0)
    k = 0
    for bb in range(NBB):
        if bb + 1 < NBB:
            hx[(bb + 1) % 2] = start_x(bb + 1)
        hx[bb % 2].wait()
        for lc in range(NLC):
            buf = k % 2
            if k >= 2:
                ho[buf].wait()
            compute(x_st.at[bb % 2], out_st.at[buf], lc * LC)
            ho[buf] = pltpu.async_copy(
                out_st.at[buf],
                out_hbm.at[:, pl.ds(lc * LC, LC), pl.ds(b0 + bb * BB, BB)],
                sem_o[buf])
            k += 1
    ho[0].wait()
    ho[1].wait()


def _make_sc_kernel(interpret=False):
    return pl.kernel(
        _sc_body,
        out_type=jax.ShapeDtypeStruct((O, L, B), jnp.float32),
        mesh=plsc.VectorSubcoreMesh(core_axis_name="c", subcore_axis_name="s",
                                    num_cores=NC, num_subcores=NS),
        compiler_params=pltpu.CompilerParams(
            use_tc_tiling_on_sc=False, needs_layout_passes=False),
        scratch_types=[
            pltpu.VMEM((O, E), jnp.float32),
            pltpu.VMEM((O, E), jnp.float32),
            pltpu.VMEM((O,), jnp.float32),
            pltpu.VMEM((16, O * O), jnp.float32),
            pltpu.VMEM((2, BB, L + 1), jnp.int32),
            pltpu.VMEM((2, O, LC, BB), jnp.float32),
            pltpu.SemaphoreType.DMA,
            pltpu.SemaphoreType.DMA,
            pltpu.SemaphoreType.DMA,
            pltpu.SemaphoreType.DMA,
        ],
        interpret=interpret,
    )


_sc_embed_mlp = _make_sc_kernel()


def kernel(x, emb_table, W, b):
    out_t = _sc_embed_mlp(emb_table, W, b, x.astype(jnp.int32))
    return jnp.transpose(out_t, (2, 1, 0))


# speedup vs baseline: 1.2440x; 1.2440x over previous
import jax
import jax.numpy as jnp
from jax import lax
from jax.experimental import pallas as pl
from jax.experimental.pallas import tpu as pltpu
from jax.experimental.pallas import tpu_sc as plsc

B, L, E, O = 16384, 200, 3, 5
NC, NS = 2, 16
NW = NC * NS
ROWS = B // NW
BB = 128
NBB = ROWS // BB
LC = 50
NLC = L // LC
LU = 2


def _build_lut(emb_v, w_v, b_v, lut_v):
    for off in (0, 16):
        p = jnp.minimum(lax.iota(jnp.int32, 16) + off, O * O - 1)
        i_v = p // O
        o_v = p % O
        acc = plsc.load_gather(b_v, [o_v])
        for d in range(E):
            dd = jnp.full((16,), d, jnp.int32)
            ev = plsc.load_gather(emb_v, [i_v, dd])
            wv = plsc.load_gather(w_v, [o_v, dd])
            acc = acc + ev * wv
        for r in range(16):
            rr = jnp.full((16,), r, jnp.int32)
            plsc.store_scatter(lut_v, [o_v, rr, i_v], acc)


def _sc_body(emb_hbm, w_hbm, b_hbm, x_hbm, out_hbm,
             emb_v, w_v, b_v, lut_v, x_st, out_st,
             sem_x0, sem_x1, sem_o0, sem_o1):
    c = lax.axis_index("c")
    s = lax.axis_index("s")
    wid = c * NS + s
    b0 = wid * ROWS

    pltpu.sync_copy(emb_hbm, emb_v)
    pltpu.sync_copy(w_hbm, w_v)
    pltpu.sync_copy(b_hbm, b_v)
    _build_lut(emb_v, w_v, b_v, lut_v)

    sem_x = (sem_x0, sem_x1)
    sem_o = (sem_o0, sem_o1)
    lane = lax.iota(jnp.int32, 16)

    def compute(xref, oref, l0):
        def l_body(it, carry):
            for u in range(LU):
                dl = it * LU + u
                l_v = jnp.full((16,), l0, jnp.int32) + dl
                for g in range(BB // 16):
                    b_v = lane + g * 16
                    xg = plsc.load_gather(xref, [b_v, l_v])
                    for o in range(O):
                        v = plsc.load_gather(lut_v.at[o], [lane, xg])
                        oref[o, dl, pl.ds(g * 16, 16)] = v
            return carry
        lax.fori_loop(0, LC // LU, l_body, 0)

    def start_x(i):
        return pltpu.async_copy(
            x_hbm.at[pl.ds(b0 + i * BB, BB), :], x_st.at[i % 2], sem_x[i % 2])

    hx = [None, None]
    ho = [None, None]
    hx[0] = start_x(0)
    k = 0
    for bb in range(NBB):
        if bb + 1 < NBB:
            hx[(bb + 1) % 2] = start_x(bb + 1)
        hx[bb % 2].wait()
        for lc in range(NLC):
            buf = k % 2
            if k >= 2:
                ho[buf].wait()
            compute(x_st.at[bb % 2], out_st.at[buf], lc * LC)
            ho[buf] = pltpu.async_copy(
                out_st.at[buf],
                out_hbm.at[:, pl.ds(lc * LC, LC), pl.ds(b0 + bb * BB, BB)],
                sem_o[buf])
            k += 1
    ho[0].wait()
    ho[1].wait()


def _make_sc_kernel(interpret=False):
    return pl.kernel(
        _sc_body,
        out_type=jax.ShapeDtypeStruct((O, L, B), jnp.float32),
        mesh=plsc.VectorSubcoreMesh(core_axis_name="c", subcore_axis_name="s",
                                    num_cores=NC, num_subcores=NS),
        compiler_params=pltpu.CompilerParams(
            use_tc_tiling_on_sc=False, needs_layout_passes=False),
        scratch_types=[
            pltpu.VMEM((O, E), jnp.float32),
            pltpu.VMEM((O, E), jnp.float32),
            pltpu.VMEM((O,), jnp.float32),
            pltpu.VMEM((O, 16, O), jnp.float32),
            pltpu.VMEM((2, BB, L), jnp.int32),
            pltpu.VMEM((2, O, LC, BB), jnp.float32),
            pltpu.SemaphoreType.DMA,
            pltpu.SemaphoreType.DMA,
            pltpu.SemaphoreType.DMA,
            pltpu.SemaphoreType.DMA,
        ],
        interpret=interpret,
    )


_sc_embed_mlp = _make_sc_kernel()


def kernel(x, emb_table, W, b):
    out_t = _sc_embed_mlp(emb_table, W, b, x.astype(jnp.int32))
    return jnp.transpose(out_t, (2, 1, 0))


# speedup vs baseline: 1.5619x; 1.2555x over previous
import jax
import jax.numpy as jnp
from jax import lax
from jax.experimental import pallas as pl
from jax.experimental.pallas import tpu as pltpu
from jax.experimental.pallas import tpu_sc as plsc

B, L, E, O = 16384, 200, 3, 5
NC, NS = 2, 16
NW = NC * NS
ROWS = B // NW
BB = 128
NBB = ROWS // BB
LC = 40
NLC = L // LC
LU = 2
LT = L // 8
BT = B // BB


def _build_lut(emb_v, w_v, b_v, lut_v):
    for off in (0, 16):
        p = jnp.minimum(lax.iota(jnp.int32, 16) + off, O * O - 1)
        i_v = p // O
        o_v = p % O
        acc = plsc.load_gather(b_v, [o_v])
        for d in range(E):
            dd = jnp.full((16,), d, jnp.int32)
            ev = plsc.load_gather(emb_v, [i_v, dd])
            wv = plsc.load_gather(w_v, [o_v, dd])
            acc = acc + ev * wv
        for r in range(16):
            rr = jnp.full((16,), r, jnp.int32)
            plsc.store_scatter(lut_v, [o_v, rr, i_v], acc)


def _sc_body(emb_hbm, w_hbm, b_hbm, x_hbm, out_hbm,
             emb_v, w_v, b_v, lut_v, x_st, out_st,
             sem_x0, sem_x1, sem_o0, sem_o1):
    c = lax.axis_index("c")
    s = lax.axis_index("s")
    wid = c * NS + s
    b0 = wid * ROWS

    pltpu.sync_copy(emb_hbm, emb_v)
    pltpu.sync_copy(w_hbm, w_v)
    pltpu.sync_copy(b_hbm, b_v)
    _build_lut(emb_v, w_v, b_v, lut_v)

    sem_x = (sem_x0, sem_x1)
    sem_o = (sem_o0, sem_o1)
    lane = lax.iota(jnp.int32, 16)

    def compute(xref, oref, l0):
        def l_body(it, carry):
            for u in range(LU):
                dl = it * LU + u
                l_v = jnp.full((16,), l0, jnp.int32) + dl
                for g in range(BB // 16):
                    b_v = lane + g * 16
                    xg = plsc.load_gather(xref, [b_v, l_v])
                    for o in range(O):
                        v = plsc.load_gather(lut_v.at[o], [lane, xg])
                        oref[o, dl // 8, dl % 8, pl.ds(g * 16, 16)] = v
            return carry
        lax.fori_loop(0, LC // LU, l_body, 0)

    def start_x(i):
        return pltpu.async_copy(
            x_hbm.at[pl.ds(b0 + i * BB, BB), :], x_st.at[i % 2], sem_x[i % 2])

    hx = [None, None]
    ho = [None, None]
    hx[0] = start_x(0)
    k = 0
    for bb in range(NBB):
        if bb + 1 < NBB:
            hx[(bb + 1) % 2] = start_x(bb + 1)
        hx[bb % 2].wait()
        for lc in range(NLC):
            buf = k % 2
            if k >= 2:
                ho[buf].wait()
            compute(x_st.at[bb % 2], out_st.at[buf], lc * LC)
            ho[buf] = pltpu.async_copy(
                out_st.at[buf],
                out_hbm.at[:, pl.ds(lc * (LC // 8), LC // 8), wid * NBB + bb],
                sem_o[buf])
            k += 1
    ho[0].wait()
    ho[1].wait()


def _make_sc_kernel(interpret=False):
    return pl.kernel(
        _sc_body,
        out_type=jax.ShapeDtypeStruct((O, LT, BT, 8, BB), jnp.float32),
        mesh=plsc.VectorSubcoreMesh(core_axis_name="c", subcore_axis_name="s",
                                    num_cores=NC, num_subcores=NS),
        compiler_params=pltpu.CompilerParams(
            use_tc_tiling_on_sc=False, needs_layout_passes=False),
        scratch_types=[
            pltpu.VMEM((O, E), jnp.float32),
            pltpu.VMEM((O, E), jnp.float32),
            pltpu.VMEM((O,), jnp.float32),
            pltpu.VMEM((O, 16, O), jnp.float32),
            pltpu.VMEM((2, BB, L), jnp.int32),
            pltpu.VMEM((2, O, LC // 8, 8, BB), jnp.float32),
            pltpu.SemaphoreType.DMA,
            pltpu.SemaphoreType.DMA,
            pltpu.SemaphoreType.DMA,
            pltpu.SemaphoreType.DMA,
        ],
        interpret=interpret,
    )


_sc_embed_mlp = _make_sc_kernel()


def kernel(x, emb_table, W, b):
    t5 = _sc_embed_mlp(emb_table, W, b, x.astype(jnp.int32))
    return jnp.transpose(t5, (2, 4, 1, 3, 0)).reshape(B, L, O)
